# baseline (device time: 177617 ns/iter reference)
import jax
import jax.numpy as jnp
from jax import lax
from jax.experimental import pallas as pl
from jax.experimental.pallas import tpu as pltpu

N_DEV = 4


def _gelu(z):
    return 0.5 * z * (1.0 + jnp.tanh(0.7978845608 * (z + 0.044715 * z * z * z)))


def kernel(A, B):
    m, k = A.shape
    _, n = B.shape

    def body(a_ref, b_ref, out_ref, comm_ref, send_sems, recv_sems):
        my = lax.axis_index("i")
        left = (my - 1) % N_DEV
        right = (my + 1) % N_DEV

        barrier_sem = pltpu.get_barrier_semaphore()
        for nbr in (left, right):
            pl.semaphore_signal(
                barrier_sem, inc=1,
                device_id=(nbr,), device_id_type=pl.DeviceIdType.MESH,
            )
        pl.semaphore_wait(barrier_sem, 2)

        partial = jnp.dot(
            a_ref[...].astype(jnp.bfloat16),
            b_ref[...].astype(jnp.bfloat16),
            preferred_element_type=jnp.float32,
        )
        out_ref[...] = partial
        comm_ref[0, :, :] = partial.astype(jnp.bfloat16)

        for h in range(N_DEV - 1):
            s = h % 2
            r = (h + 1) % 2
            rdma = pltpu.make_async_remote_copy(
                src_ref=comm_ref.at[s],
                dst_ref=comm_ref.at[r],
                send_sem=send_sems.at[s],
                recv_sem=recv_sems.at[r],
                device_id=(right,),
                device_id_type=pl.DeviceIdType.MESH,
            )
            rdma.start()
            rdma.wait()
            out_ref[...] += comm_ref[r, :, :].astype(jnp.float32)

        out_ref[...] = _gelu(out_ref[...])

    return pl.pallas_call(
        body,
        out_shape=jax.ShapeDtypeStruct((m, n), jnp.float32),
        in_specs=[
            pl.BlockSpec(memory_space=pltpu.VMEM),
            pl.BlockSpec(memory_space=pltpu.VMEM),
        ],
        out_specs=pl.BlockSpec(memory_space=pltpu.VMEM),
        scratch_shapes=[
            pltpu.VMEM((2, m, n), jnp.bfloat16),
            pltpu.SemaphoreType.DMA((2,)),
            pltpu.SemaphoreType.DMA((2,)),
        ],
        compiler_params=pltpu.CompilerParams(collective_id=0),
    )(A, B)


# device time: 60433 ns/iter; 2.9391x vs baseline; 2.9391x over previous
import jax
import jax.numpy as jnp
from jax import lax
from jax.experimental import pallas as pl
from jax.experimental.pallas import tpu as pltpu

N_DEV = 4
BF = jnp.bfloat16
F32 = jnp.float32


def _gelu(z):
    return 0.5 * z * (1.0 + jnp.tanh(0.7978845608 * (z + 0.044715 * z * z * z)))


def kernel(A, B):
    m, k = A.shape
    _, n = B.shape
    h = n // 2
    q = m // 4
    hm = m // 2

    def body(a_ref, b_ref, out_ref, p_ref, r1, r2, send_sems, recv_sems):
        my = lax.axis_index("i")
        yp = my ^ 1
        xp = my ^ 3
        role_x = (my >> 1) & 1
        role_y = (my & 1) ^ role_x

        barrier_sem = pltpu.get_barrier_semaphore()
        for nbr in (yp, xp):
            pl.semaphore_signal(
                barrier_sem, inc=1,
                device_id=(nbr,), device_id_type=pl.DeviceIdType.MESH,
            )
        pl.semaphore_wait(barrier_sem, 2)

        partial = jnp.dot(
            a_ref[...].astype(BF), b_ref[...].astype(BF),
            preferred_element_type=F32,
        )
        p_ref[...] = partial.astype(BF)

        def exchange(src, dst, idx, partner):
            rdma = pltpu.make_async_remote_copy(
                src_ref=src, dst_ref=dst,
                send_sem=send_sems.at[idx], recv_sem=recv_sems.at[idx],
                device_id=(partner,), device_id_type=pl.DeviceIdType.MESH,
            )
            rdma.start()
            return rdma

        def add_bf16(dst_ref_slice, recv):
            return (dst_ref_slice.astype(F32) + recv.astype(F32)).astype(BF)

        e0 = exchange(p_ref.at[pl.ds((1 - role_y) * hm, hm), pl.ds(0, h)],
                      r1.at[0], 0, yp)
        e1 = exchange(p_ref.at[pl.ds((1 - role_x) * hm, hm), pl.ds(h, h)],
                      r1.at[1], 1, xp)
        e0.wait()
        e1.wait()
        o_h0 = role_y * hm
        o_h1 = role_x * hm
        p_ref[pl.ds(o_h0, hm), pl.ds(0, h)] = add_bf16(
            p_ref[pl.ds(o_h0, hm), pl.ds(0, h)], r1[0])
        p_ref[pl.ds(o_h1, hm), pl.ds(h, h)] = add_bf16(
            p_ref[pl.ds(o_h1, hm), pl.ds(h, h)], r1[1])

        e0 = exchange(p_ref.at[pl.ds(o_h0 + (1 - role_x) * q, q), pl.ds(0, h)],
                      r2.at[0], 2, xp)
        e1 = exchange(p_ref.at[pl.ds(o_h1 + (1 - role_y) * q, q), pl.ds(h, h)],
                      r2.at[1], 3, yp)
        e0.wait()
        e1.wait()
        q_h0 = o_h0 + role_x * q
        q_h1 = o_h1 + role_y * q
        z0 = p_ref[pl.ds(q_h0, q), pl.ds(0, h)].astype(F32) + r2[0].astype(F32)
        z1 = p_ref[pl.ds(q_h1, q), pl.ds(h, h)].astype(F32) + r2[1].astype(F32)

        out_ref[pl.ds(q_h0, q), pl.ds(0, h)] = _gelu(z0).astype(BF)
        out_ref[pl.ds(q_h1, q), pl.ds(h, h)] = _gelu(z1).astype(BF)

        e0 = exchange(out_ref.at[pl.ds(q_h0, q), pl.ds(0, h)],
                      out_ref.at[pl.ds(q_h0, q), pl.ds(0, h)], 4, xp)
        e1 = exchange(out_ref.at[pl.ds(q_h1, q), pl.ds(h, h)],
                      out_ref.at[pl.ds(q_h1, q), pl.ds(h, h)], 5, yp)
        e0.wait()
        e1.wait()

        e0 = exchange(out_ref.at[pl.ds(o_h0, hm), pl.ds(0, h)],
                      out_ref.at[pl.ds(o_h0, hm), pl.ds(0, h)], 6, yp)
        e1 = exchange(out_ref.at[pl.ds(o_h1, hm), pl.ds(h, h)],
                      out_ref.at[pl.ds(o_h1, hm), pl.ds(h, h)], 7, xp)
        e0.wait()
        e1.wait()

    return pl.pallas_call(
        body,
        out_shape=jax.ShapeDtypeStruct((m, n), BF),
        in_specs=[
            pl.BlockSpec(memory_space=pltpu.VMEM),
            pl.BlockSpec(memory_space=pltpu.VMEM),
        ],
        out_specs=pl.BlockSpec(memory_space=pltpu.VMEM),
        scratch_shapes=[
            pltpu.VMEM((m, n), BF),
            pltpu.VMEM((2, hm, h), BF),
            pltpu.VMEM((2, q, h), BF),
            pltpu.SemaphoreType.DMA((8,)),
            pltpu.SemaphoreType.DMA((8,)),
        ],
        compiler_params=pltpu.CompilerParams(collective_id=0),
    )(A, B)


# device time: 57974 ns/iter; 3.0637x vs baseline; 1.0424x over previous
import jax
import jax.numpy as jnp
from jax import lax
from jax.experimental import pallas as pl
from jax.experimental.pallas import tpu as pltpu

N_DEV = 4
BF = jnp.bfloat16
F32 = jnp.float32


def _gelu(z):
    return 0.5 * z * (1.0 + jnp.tanh(0.7978845608 * (z + 0.044715 * z * z * z)))


def kernel(A, B):
    m, k = A.shape
    _, n = B.shape
    h = n // 2
    q = m // 4
    hm = m // 2

    def body(a_ref, b_ref, out_ref, p_ref, r1, r2, send_sems, recv_sems):
        my = lax.axis_index("i")
        yp = my ^ 1
        xp = my ^ 3
        role_x = (my >> 1) & 1
        role_y = (my & 1) ^ role_x

        barrier_sem = pltpu.get_barrier_semaphore()
        for nbr in (yp, xp):
            pl.semaphore_signal(
                barrier_sem, inc=1,
                device_id=(nbr,), device_id_type=pl.DeviceIdType.MESH,
            )
        pl.semaphore_wait(barrier_sem, 2)

        def exchange(src, dst, idx, partner):
            rdma = pltpu.make_async_remote_copy(
                src_ref=src, dst_ref=dst,
                send_sem=send_sems.at[idx], recv_sem=recv_sems.at[idx],
                device_id=(partner,), device_id_type=pl.DeviceIdType.MESH,
            )
            rdma.start()
            return rdma

        def quad(rows_off, cols_off):
            return jnp.dot(
                a_ref[pl.ds(rows_off, hm), :].astype(BF),
                b_ref[:, pl.ds(cols_off, h)].astype(BF),
                preferred_element_type=F32,
            ).astype(BF)

        o_h0 = role_y * hm
        o_h1 = role_x * hm
        s_h0 = (1 - role_y) * hm
        s_h1 = (1 - role_x) * hm

        p_ref[pl.ds(s_h0, hm), pl.ds(0, h)] = quad(s_h0, 0)
        e0 = exchange(p_ref.at[pl.ds(s_h0, hm), pl.ds(0, h)], r1.at[0], 0, yp)
        p_ref[pl.ds(s_h1, hm), pl.ds(h, h)] = quad(s_h1, h)
        e1 = exchange(p_ref.at[pl.ds(s_h1, hm), pl.ds(h, h)], r1.at[1], 1, xp)
        p_ref[pl.ds(o_h0, hm), pl.ds(0, h)] = quad(o_h0, 0)
        p_ref[pl.ds(o_h1, hm), pl.ds(h, h)] = quad(o_h1, h)

        def add_bf16(x, y):
            return (x.astype(F32) + y.astype(F32)).astype(BF)

        q0s = o_h0 + (1 - role_x) * q
        q0k = o_h0 + role_x * q
        e0.wait()
        p_ref[pl.ds(q0s, q), pl.ds(0, h)] = add_bf16(
            p_ref[pl.ds(q0s, q), pl.ds(0, h)],
            r1[0, pl.ds((1 - role_x) * q, q), :])
        e2 = exchange(p_ref.at[pl.ds(q0s, q), pl.ds(0, h)], r2.at[0], 2, xp)
        p_ref[pl.ds(q0k, q), pl.ds(0, h)] = add_bf16(
            p_ref[pl.ds(q0k, q), pl.ds(0, h)],
            r1[0, pl.ds(role_x * q, q), :])

        q1s = o_h1 + (1 - role_y) * q
        q1k = o_h1 + role_y * q
        e1.wait()
        p_ref[pl.ds(q1s, q), pl.ds(h, h)] = add_bf16(
            p_ref[pl.ds(q1s, q), pl.ds(h, h)],
            r1[1, pl.ds((1 - role_y) * q, q), :])
        e3 = exchange(p_ref.at[pl.ds(q1s, q), pl.ds(h, h)], r2.at[1], 3, yp)
        p_ref[pl.ds(q1k, q), pl.ds(h, h)] = add_bf16(
            p_ref[pl.ds(q1k, q), pl.ds(h, h)],
            r1[1, pl.ds(role_y * q, q), :])

        e2.wait()
        z0 = p_ref[pl.ds(q0k, q), pl.ds(0, h)].astype(F32) + r2[0].astype(F32)
        out_ref[pl.ds(q0k, q), pl.ds(0, h)] = _gelu(z0).astype(BF)
        e4 = exchange(out_ref.at[pl.ds(q0k, q), pl.ds(0, h)],
                      out_ref.at[pl.ds(q0k, q), pl.ds(0, h)], 4, xp)

        e3.wait()
        z1 = p_ref[pl.ds(q1k, q), pl.ds(h, h)].astype(F32) + r2[1].astype(F32)
        out_ref[pl.ds(q1k, q), pl.ds(h, h)] = _gelu(z1).astype(BF)
        e5 = exchange(out_ref.at[pl.ds(q1k, q), pl.ds(h, h)],
                      out_ref.at[pl.ds(q1k, q), pl.ds(h, h)], 5, yp)

        e4.wait()
        e6 = exchange(out_ref.at[pl.ds(o_h0, hm), pl.ds(0, h)],
                      out_ref.at[pl.ds(o_h0, hm), pl.ds(0, h)], 6, yp)
        e5.wait()
        e7 = exchange(out_ref.at[pl.ds(o_h1, hm), pl.ds(h, h)],
                      out_ref.at[pl.ds(o_h1, hm), pl.ds(h, h)], 7, xp)
        e6.wait()
        e7.wait()

    return pl.pallas_call(
        body,
        out_shape=jax.ShapeDtypeStruct((m, n), BF),
        in_specs=[
            pl.BlockSpec(memory_space=pltpu.VMEM),
            pl.BlockSpec(memory_space=pltpu.VMEM),
        ],
        out_specs=pl.BlockSpec(memory_space=pltpu.VMEM),
        scratch_shapes=[
            pltpu.VMEM((m, n), BF),
            pltpu.VMEM((2, hm, h), BF),
            pltpu.VMEM((2, q, h), BF),
            pltpu.SemaphoreType.DMA((8,)),
            pltpu.SemaphoreType.DMA((8,)),
        ],
        compiler_params=pltpu.CompilerParams(collective_id=0),
    )(A, B)


# device time: 51624 ns/iter; 3.4406x vs baseline; 1.1230x over previous
import jax
import jax.numpy as jnp
from jax import lax
from jax.experimental import pallas as pl
from jax.experimental.pallas import tpu as pltpu

N_DEV = 4
N_CHUNK = 4
BF = jnp.bfloat16
F32 = jnp.float32


def _gelu(z):
    return 0.5 * z * (1.0 + jnp.tanh(0.7978845608 * (z + 0.044715 * z * z * z)))


def kernel(A, B):
    m, k = A.shape
    _, n = B.shape
    w = n // N_CHUNK
    q = m // 4
    hm = m // 2

    def body(a_ref, b_ref, out_ref, p_ref, r1, r2, send_sems, recv_sems):
        my = lax.axis_index("i")
        yp = my ^ 1
        xp = my ^ 3
        role_x = (my >> 1) & 1
        role_y = (my & 1) ^ role_x

        barrier_sem = pltpu.get_barrier_semaphore()
        for nbr in (yp, xp):
            pl.semaphore_signal(
                barrier_sem, inc=1,
                device_id=(nbr,), device_id_type=pl.DeviceIdType.MESH,
            )
        pl.semaphore_wait(barrier_sem, 2)

        def exchange(src, dst, idx, partner):
            rdma = pltpu.make_async_remote_copy(
                src_ref=src, dst_ref=dst,
                send_sem=send_sems.at[idx], recv_sem=recv_sems.at[idx],
                device_id=(partner,), device_id_type=pl.DeviceIdType.MESH,
            )
            rdma.start()
            return rdma

        def add_bf16(x, y):
            return (x.astype(F32) + y.astype(F32)).astype(BF)

        def cfg(t):
            c = t * w
            if t < 2:
                rl1, rl2, p1, p2 = role_y, role_x, yp, xp
            else:
                rl1, rl2, p1, p2 = role_x, role_y, xp, yp
            o1 = rl1 * hm
            s1 = (1 - rl1) * hm
            qs = o1 + (1 - rl2) * q
            qk = o1 + rl2 * q
            return dict(c=c, p1=p1, p2=p2, rl2=rl2, o1=o1, s1=s1, qs=qs, qk=qk)

        cs = [cfg(t) for t in range(N_CHUNK)]
        order = [0, 2, 1, 3]

        def quad(rows_off, cols_off):
            return jnp.dot(
                a_ref[pl.ds(rows_off, hm), :].astype(BF),
                b_ref[:, pl.ds(cols_off, w)].astype(BF),
                preferred_element_type=F32,
            ).astype(BF)

        e1s = {}
        with jax.named_scope("mm_send"):
            for t in order:
                c = cs[t]
                p_ref[pl.ds(c["s1"], hm), pl.ds(c["c"], w)] = quad(c["s1"], c["c"])
                e1s[t] = exchange(
                    p_ref.at[pl.ds(c["s1"], hm), pl.ds(c["c"], w)],
                    r1.at[t], t, c["p1"])
        with jax.named_scope("mm_keep"):
            for t in order:
                c = cs[t]
                p_ref[pl.ds(c["o1"], hm), pl.ds(c["c"], w)] = quad(c["o1"], c["c"])

        e2s = {}
        for t in order:
            c = cs[t]
            with jax.named_scope(f"p1_wait_{t}"):
                e1s[t].wait()
            with jax.named_scope(f"p1_add_{t}"):
                p_ref[pl.ds(c["qs"], q), pl.ds(c["c"], w)] = add_bf16(
                    p_ref[pl.ds(c["qs"], q), pl.ds(c["c"], w)],
                    r1[t, pl.ds((1 - c["rl2"]) * q, q), :])
                e2s[t] = exchange(
                    p_ref.at[pl.ds(c["qs"], q), pl.ds(c["c"], w)],
                    r2.at[t], 4 + t, c["p2"])
                p_ref[pl.ds(c["qk"], q), pl.ds(c["c"], w)] = add_bf16(
                    p_ref[pl.ds(c["qk"], q), pl.ds(c["c"], w)],
                    r1[t, pl.ds(c["rl2"] * q, q), :])

        e3s = {}
        for t in order:
            c = cs[t]
            with jax.named_scope(f"p2_wait_{t}"):
                e2s[t].wait()
            with jax.named_scope(f"gelu_{t}"):
                z = (p_ref[pl.ds(c["qk"], q), pl.ds(c["c"], w)].astype(F32)
                     + r2[t].astype(F32))
                out_ref[pl.ds(c["qk"], q), pl.ds(c["c"], w)] = _gelu(z).astype(BF)
                e3s[t] = exchange(
                    out_ref.at[pl.ds(c["qk"], q), pl.ds(c["c"], w)],
                    out_ref.at[pl.ds(c["qk"], q), pl.ds(c["c"], w)],
                    8 + t, c["p2"])

        e4s = {}
        for t in order:
            c = cs[t]
            with jax.named_scope(f"p3_wait_{t}"):
                e3s[t].wait()
            e4s[t] = exchange(
                out_ref.at[pl.ds(c["o1"], hm), pl.ds(c["c"], w)],
                out_ref.at[pl.ds(c["o1"], hm), pl.ds(c["c"], w)],
                12 + t, c["p1"])
        with jax.named_scope("p4_wait"):
            for t in order:
                e4s[t].wait()

    return pl.pallas_call(
        body,
        out_shape=jax.ShapeDtypeStruct((m, n), BF),
        in_specs=[
            pl.BlockSpec(memory_space=pltpu.VMEM),
            pl.BlockSpec(memory_space=pltpu.VMEM),
        ],
        out_specs=pl.BlockSpec(memory_space=pltpu.VMEM),
        scratch_shapes=[
            pltpu.VMEM((m, n), BF),
            pltpu.VMEM((N_CHUNK, hm, w), BF),
            pltpu.VMEM((N_CHUNK, q, w), BF),
            pltpu.SemaphoreType.DMA((16,)),
            pltpu.SemaphoreType.DMA((16,)),
        ],
        compiler_params=pltpu.CompilerParams(collective_id=0),
    )(A, B)
